# baseline (speedup 1.0000x reference)
import jax
import jax.numpy as jnp
from jax.experimental import pallas as pl
from jax.experimental.pallas import tpu as pltpu
from functools import partial

_THEA = 0.5
_DELTA = 1.5
_K = 8
_EPS = 1e-12


def _dl_body(xa_ref, xb_ref, t_ref, out_ref,
             sums_ref, counts_ref, means_ref, mnorm_ref, accb_ref,
             *, half_tiles):
    s = pl.program_id(0)
    ht = half_tiles

    @pl.when(s == 0)
    def _init():
        sums_ref[...] = jnp.zeros_like(sums_ref)
        counts_ref[...] = jnp.zeros_like(counts_ref)
        accb_ref[...] = jnp.zeros_like(accb_ref)

    t = t_ref[0]
    kidx = jax.lax.broadcasted_iota(jnp.int32, (_K, t.shape[1]), 0)
    maskf = (kidx == t).astype(jnp.bfloat16)

    def _pass_a(x_ref):
        x = x_ref[...]
        sums_ref[...] += jax.lax.dot_general(
            maskf, x, (((1,), (1,)), ((), ())),
            preferred_element_type=jnp.float32)
        counts_ref[...] += jnp.sum(maskf, axis=1, keepdims=True)

    @pl.when(s < ht)
    def _a0():
        _pass_a(xa_ref)

    @pl.when((s >= ht) & (s < 2 * ht))
    def _a1():
        _pass_a(xb_ref)

    @pl.when(s == 2 * ht)
    def _mk_means():
        m = sums_ref[...] / counts_ref[...]
        means_ref[...] = m
        mnorm_ref[...] = jnp.sum(m * m, axis=1, keepdims=True)

    def _pass_b(x_ref):
        x = x_ref[...]
        m = means_ref[...].astype(jnp.bfloat16)
        msel = jax.lax.dot_general(
            m, maskf, (((0,), (0,)), ((), ())),
            preferred_element_type=jnp.float32)
        diff = x - msel.astype(jnp.bfloat16)
        sq = diff * diff
        ones_c = jnp.ones((1, x.shape[0]), dtype=jnp.bfloat16)
        dsq = jax.lax.dot_general(
            ones_c, sq, (((1,), (0,)), ((), ())),
            preferred_element_type=jnp.float32)
        d = jnp.sqrt(dsq + _EPS)
        r = jnp.maximum(d - _THEA, 0.0)
        r2 = r * r
        accb_ref[...] += jnp.sum(maskf.astype(jnp.float32) * r2,
                                 axis=1, keepdims=True)

    @pl.when((s >= 2 * ht) & (s < 3 * ht))
    def _b0():
        _pass_b(xa_ref)

    @pl.when(s >= 3 * ht)
    def _b1():
        _pass_b(xb_ref)

    @pl.when(s == 4 * ht - 1)
    def _epilogue():
        counts = counts_ref[...]
        m = means_ref[...]
        mnorm = mnorm_ref[...]
        loss_var = jnp.sum(accb_ref[...] / counts) / _K
        g = jax.lax.dot_general(
            m, m, (((1,), (1,)), ((), ())),
            preferred_element_type=jnp.float32)
        ri = jax.lax.broadcasted_iota(jnp.int32, (_K, _K), 0)
        ci = jax.lax.broadcasted_iota(jnp.int32, (_K, _K), 1)
        eye = (ri == ci).astype(jnp.float32)
        diag_col = jnp.sum(g * eye, axis=1, keepdims=True)
        diag_row = jnp.sum(g * eye, axis=0, keepdims=True)
        dist_sq = diag_col + diag_row - 2.0 * g
        dist = jnp.sqrt(dist_sq + eye)
        pen = jnp.maximum(2.0 * _DELTA - dist, 0.0) ** 2 * (1.0 - eye)
        loss_dis = jnp.sum(pen) / (_K * (_K - 1))
        loss_reg = jnp.sum(jnp.sqrt(mnorm + _EPS)) / _K
        out_ref[...] = jnp.reshape(loss_var + loss_dis + 0.001 * loss_reg,
                                   (1, 1))


def kernel(predict, target):
    n, c, h, w = predict.shape
    pix = h * w
    p_blk = 131072 if pix % 131072 == 0 else pix
    j_tiles = pix // p_blk
    nh = n // 2
    ht = nh * j_tiles

    xa = predict[:nh].astype(jnp.bfloat16).reshape(nh * c, pix)
    xb = predict[nh:].astype(jnp.bfloat16).reshape(nh * c, pix)
    t3 = target.reshape(2 * ht, 1, p_blk)

    def xa_map(s):
        idx = jax.lax.select(
            s < ht, s,
            jax.lax.select(s < 2 * ht, ht - 1,
                           jax.lax.select(s < 3 * ht, s - 2 * ht, ht - 1)))
        return idx // j_tiles, idx % j_tiles

    def xb_map(s):
        idx = jax.lax.select(
            s < ht, 0,
            jax.lax.select(s < 2 * ht, s - ht,
                           jax.lax.select(s < 3 * ht, 0, s - 3 * ht)))
        return idx // j_tiles, idx % j_tiles

    def t_map(s):
        return jax.lax.rem(s, 2 * ht), 0, 0

    out = pl.pallas_call(
        partial(_dl_body, half_tiles=ht),
        grid=(4 * ht,),
        in_specs=[
            pl.BlockSpec((c, p_blk), xa_map),
            pl.BlockSpec((c, p_blk), xb_map),
            pl.BlockSpec((1, 1, p_blk), t_map),
        ],
        out_specs=pl.BlockSpec((1, 1), lambda s: (0, 0)),
        out_shape=jax.ShapeDtypeStruct((1, 1), jnp.float32),
        scratch_shapes=[
            pltpu.VMEM((_K, c), jnp.float32),
            pltpu.VMEM((_K, 1), jnp.float32),
            pltpu.VMEM((_K, c), jnp.float32),
            pltpu.VMEM((_K, 1), jnp.float32),
            pltpu.VMEM((_K, 1), jnp.float32),
        ],
        compiler_params=pltpu.CompilerParams(
            dimension_semantics=("arbitrary",)),
    )(xa, xb, t3)
    return out[0, 0]


# speedup vs baseline: 1.2655x; 1.2655x over previous
import jax
import jax.numpy as jnp
from jax.experimental import pallas as pl
from jax.experimental.pallas import tpu as pltpu
from functools import partial

_THEA = 0.5
_DELTA = 1.5
_K = 8
_EPS = 1e-12


def _dl_body(x_ref, t_ref, out_ref,
             sums_ref, counts_ref, means_ref, mnorm_ref, accb_ref,
             *, n_tiles):
    s = pl.program_id(0)

    @pl.when(s == 0)
    def _init():
        sums_ref[...] = jnp.zeros_like(sums_ref)
        counts_ref[...] = jnp.zeros_like(counts_ref)
        accb_ref[...] = jnp.zeros_like(accb_ref)

    x = x_ref[...]
    t = t_ref[0]
    kidx = jax.lax.broadcasted_iota(jnp.int32, (_K, x.shape[1]), 0)
    maskf = (kidx == t).astype(jnp.bfloat16)

    @pl.when(s < n_tiles)
    def _pass_a():
        sums_ref[...] += jax.lax.dot_general(
            maskf, x, (((1,), (1,)), ((), ())),
            preferred_element_type=jnp.float32)
        counts_ref[...] += jnp.sum(maskf, axis=1, keepdims=True)

    @pl.when(s == n_tiles)
    def _mk_means():
        m = sums_ref[...] / counts_ref[...]
        means_ref[...] = m
        mnorm_ref[...] = jnp.sum(m * m, axis=1, keepdims=True)

    @pl.when(s >= n_tiles)
    def _pass_b():
        m = means_ref[...].astype(jnp.bfloat16)
        msel = jax.lax.dot_general(
            m, maskf, (((0,), (0,)), ((), ())),
            preferred_element_type=jnp.float32)
        diff = x - msel.astype(jnp.bfloat16)
        sq = diff * diff
        ones_c = jnp.ones((1, x.shape[0]), dtype=jnp.bfloat16)
        dsq = jax.lax.dot_general(
            ones_c, sq, (((1,), (0,)), ((), ())),
            preferred_element_type=jnp.float32)
        d = jnp.sqrt(dsq + _EPS)
        r = jnp.maximum(d - _THEA, 0.0)
        r2 = r * r
        accb_ref[...] += jnp.sum(maskf.astype(jnp.float32) * r2,
                                 axis=1, keepdims=True)

    @pl.when(s == 2 * n_tiles - 1)
    def _epilogue():
        counts = counts_ref[...]
        m = means_ref[...]
        mnorm = mnorm_ref[...]
        loss_var = jnp.sum(accb_ref[...] / counts) / _K
        g = jax.lax.dot_general(
            m, m, (((1,), (1,)), ((), ())),
            preferred_element_type=jnp.float32)
        ri = jax.lax.broadcasted_iota(jnp.int32, (_K, _K), 0)
        ci = jax.lax.broadcasted_iota(jnp.int32, (_K, _K), 1)
        eye = (ri == ci).astype(jnp.float32)
        diag_col = jnp.sum(g * eye, axis=1, keepdims=True)
        diag_row = jnp.sum(g * eye, axis=0, keepdims=True)
        dist_sq = diag_col + diag_row - 2.0 * g
        dist = jnp.sqrt(dist_sq + eye)
        pen = jnp.maximum(2.0 * _DELTA - dist, 0.0) ** 2 * (1.0 - eye)
        loss_dis = jnp.sum(pen) / (_K * (_K - 1))
        loss_reg = jnp.sum(jnp.sqrt(mnorm + _EPS)) / _K
        out_ref[...] = jnp.reshape(loss_var + loss_dis + 0.001 * loss_reg,
                                   (1, 1))


def kernel(predict, target):
    n, c, h, w = predict.shape
    pix = h * w
    p_blk = 131072 if pix % 131072 == 0 else pix
    j_tiles = pix // p_blk
    n_tiles = n * j_tiles

    x2 = predict.astype(jnp.bfloat16).reshape(n * c, pix)
    t3 = target.reshape(n_tiles, 1, p_blk)

    def x_map(s):
        tile = jax.lax.rem(s, n_tiles)
        return tile // j_tiles, tile % j_tiles

    def t_map(s):
        return jax.lax.rem(s, n_tiles), 0, 0

    out = pl.pallas_call(
        partial(_dl_body, n_tiles=n_tiles),
        grid=(2 * n_tiles,),
        in_specs=[
            pl.BlockSpec((c, p_blk), x_map),
            pl.BlockSpec((1, 1, p_blk), t_map),
        ],
        out_specs=pl.BlockSpec((1, 1), lambda s: (0, 0)),
        out_shape=jax.ShapeDtypeStruct((1, 1), jnp.float32),
        scratch_shapes=[
            pltpu.VMEM((_K, c), jnp.float32),
            pltpu.VMEM((_K, 1), jnp.float32),
            pltpu.VMEM((_K, c), jnp.float32),
            pltpu.VMEM((_K, 1), jnp.float32),
            pltpu.VMEM((_K, 1), jnp.float32),
        ],
        compiler_params=pltpu.CompilerParams(
            dimension_semantics=("arbitrary",)),
    )(x2, t3)
    return out[0, 0]
